# baseline (speedup 1.0000x reference)
import functools

import jax
import jax.numpy as jnp
from jax import lax
from jax.experimental import pallas as pl
from jax.experimental.pallas import tpu as pltpu
from jax.experimental.pallas import tpu_sc as plsc

_H = 8192
_W = 8192
_MAX_OPS = 35
_NC = 2
_NS = 16
_NW = _NC * _NS
_RPW = _H // _NW
_RC = 8
_NCH = _RPW // _RC


def _body(params_hbm, zeros_hbm, onehot_hbm, mask_hbm, scal_hbm,
          zbuf, pbuf, obuf, wbuf, sem):
    cid = lax.axis_index("c")
    sid = lax.axis_index("s")
    wid = sid * _NC + cid
    base = wid * _RPW

    pltpu.sync_copy(params_hbm, pbuf)

    lane = lax.iota(jnp.int32, 16)
    lim = jnp.where(
        lane == 0,
        _MAX_OPS - 1,
        jnp.where((lane == 1) | (lane == 2), _H - 1, 0),
    )
    vals = pbuf[...]
    clipped = jnp.clip(vals, 0, lim)
    obuf[...] = clipped

    @pl.when(wid == 0)
    def _():
        pltpu.sync_copy(obuf, scal_hbm)

    r = jnp.clip(vals[1], 0, _H - 1)
    c = jnp.clip(vals[2], 0, _W - 1)

    pltpu.sync_copy(zeros_hbm, zbuf)
    copies = [
        pltpu.async_copy(
            zbuf, mask_hbm.at[pl.ds(base + ck * _RC, _RC), :], sem
        )
        for ck in range(_NCH)
    ]
    for cp in copies:
        cp.wait()

    wb = (c // 64) * 64

    @pl.when((r >= base) & (r < base + _RPW))
    def _():
        pltpu.sync_copy(onehot_hbm, wbuf)
        pltpu.sync_copy(wbuf, mask_hbm.at[r, pl.ds(wb, 64)])


_point_mask = functools.partial(
    pl.kernel,
    out_type=(
        jax.ShapeDtypeStruct((_H, _W), jnp.bool_),
        jax.ShapeDtypeStruct((16,), jnp.int32),
    ),
    mesh=plsc.VectorSubcoreMesh(core_axis_name="c", subcore_axis_name="s"),
    scratch_types=[
        pltpu.VMEM((_RC, _W), jnp.bool_),
        pltpu.VMEM((16,), jnp.int32),
        pltpu.VMEM((16,), jnp.int32),
        pltpu.VMEM((64,), jnp.bool_),
        pltpu.SemaphoreType.DMA,
    ],
)(_body)


def kernel(operation, action_type, row, col, grid_height, grid_width):
    head = jnp.stack(
        [
            jnp.asarray(operation, jnp.int32),
            jnp.asarray(row, jnp.int32),
            jnp.asarray(col, jnp.int32),
            jnp.asarray(action_type, jnp.int32),
        ]
    )
    params = jnp.concatenate([head, jnp.zeros((12,), jnp.int32)])
    zeros_block = jnp.zeros((_RC, _W), jnp.bool_)
    vc = jnp.clip(jnp.asarray(col, jnp.int32), 0, _W - 1)
    onehot = jnp.arange(64, dtype=jnp.int32) == (vc % 64)
    mask, scal = _point_mask(params, zeros_block, onehot)
    return (mask, scal[0], scal[3], scal[1], scal[2])


# speedup vs baseline: 2.2252x; 2.2252x over previous
import functools

import jax
import jax.numpy as jnp
from jax import lax
from jax.experimental import pallas as pl
from jax.experimental.pallas import tpu as pltpu
from jax.experimental.pallas import tpu_sc as plsc

_H = 8192
_W = 8192
_MAX_OPS = 35
_NC = 2
_NS = 16
_NW = _NC * _NS
_RPW = _H // _NW
_RC = 32
_NCH = _RPW // _RC


def _body(params_hbm, zeros_hbm, onehot_hbm, mask_hbm, scal_hbm,
          zbuf, pbuf, obuf, wbuf, sem):
    cid = lax.axis_index("c")
    sid = lax.axis_index("s")
    wid = sid * _NC + cid
    base = wid * _RPW

    pltpu.sync_copy(params_hbm, pbuf)

    lane = lax.iota(jnp.int32, 16)
    lim = jnp.where(
        lane == 0,
        _MAX_OPS - 1,
        jnp.where((lane == 1) | (lane == 2), _H - 1, 0),
    )
    vals = pbuf[...]
    clipped = jnp.clip(vals, 0, lim)
    obuf[...] = clipped

    @pl.when(wid == 0)
    def _():
        pltpu.sync_copy(obuf, scal_hbm)

    r = jnp.clip(vals[1], 0, _H - 1)
    c = jnp.clip(vals[2], 0, _W - 1)

    pltpu.sync_copy(zeros_hbm, zbuf)
    copies = [
        pltpu.async_copy(
            zbuf, mask_hbm.at[pl.ds(base + ck * _RC, _RC), :], sem
        )
        for ck in range(_NCH)
    ]
    for cp in copies:
        cp.wait()

    rb = (r // 8) * 8
    cb = (c // 128) * 128

    @pl.when((r >= base) & (r < base + _RPW))
    def _():
        pltpu.sync_copy(onehot_hbm, wbuf)
        pltpu.sync_copy(wbuf, mask_hbm.at[pl.ds(rb, 8), pl.ds(cb, 128)])


_point_mask = functools.partial(
    pl.kernel,
    out_type=(
        jax.ShapeDtypeStruct((_H, _W), jnp.int8),
        jax.ShapeDtypeStruct((16,), jnp.int32),
    ),
    mesh=plsc.VectorSubcoreMesh(core_axis_name="c", subcore_axis_name="s"),
    scratch_types=[
        pltpu.VMEM((_RC, _W), jnp.int8),
        pltpu.VMEM((16,), jnp.int32),
        pltpu.VMEM((16,), jnp.int32),
        pltpu.VMEM((8, 128), jnp.int8),
        pltpu.SemaphoreType.DMA,
    ],
)(_body)


def kernel(operation, action_type, row, col, grid_height, grid_width):
    head = jnp.stack(
        [
            jnp.asarray(operation, jnp.int32),
            jnp.asarray(row, jnp.int32),
            jnp.asarray(col, jnp.int32),
            jnp.asarray(action_type, jnp.int32),
        ]
    )
    params = jnp.concatenate([head, jnp.zeros((12,), jnp.int32)])
    zeros_block = jnp.zeros((_RC, _W), jnp.int8)
    vr = jnp.clip(jnp.asarray(row, jnp.int32), 0, _H - 1)
    vc = jnp.clip(jnp.asarray(col, jnp.int32), 0, _W - 1)
    onehot = (
        (jnp.arange(8, dtype=jnp.int32)[:, None] == (vr % 8))
        & (jnp.arange(128, dtype=jnp.int32)[None, :] == (vc % 128))
    ).astype(jnp.int8)
    mask, scal = _point_mask(params, zeros_block, onehot)
    return (mask, scal[0], scal[3], scal[1], scal[2])


# speedup vs baseline: 2.2358x; 1.0048x over previous
import functools

import jax
import jax.numpy as jnp
from jax import lax
from jax.experimental import pallas as pl
from jax.experimental.pallas import tpu as pltpu
from jax.experimental.pallas import tpu_sc as plsc

_H = 8192
_W = 8192
_MAX_OPS = 35
_NC = 2
_NS = 16
_NW = _NC * _NS
_RPW = _H // _NW
_RC = 8
_NCH = _RPW // _RC


def _body(params_hbm, zeros_hbm, onehot_hbm, mask_hbm, scal_hbm,
          zbuf, pbuf, obuf, wbuf, sem):
    cid = lax.axis_index("c")
    sid = lax.axis_index("s")
    wid = sid * _NC + cid
    base = wid * _RPW

    pltpu.sync_copy(params_hbm, pbuf)

    lane = lax.iota(jnp.int32, 16)
    lim = jnp.where(
        lane == 0,
        _MAX_OPS - 1,
        jnp.where((lane == 1) | (lane == 2), _H - 1, 0),
    )
    vals = pbuf[...]
    clipped = jnp.clip(vals, 0, lim)
    obuf[...] = clipped

    @pl.when(wid == 0)
    def _():
        pltpu.sync_copy(obuf, scal_hbm)

    r = jnp.clip(vals[1], 0, _H - 1)
    c = jnp.clip(vals[2], 0, _W - 1)

    pltpu.sync_copy(zeros_hbm, zbuf)
    copies = [
        pltpu.async_copy(
            zbuf, mask_hbm.at[pl.ds(base + ck * _RC, _RC), :], sem
        )
        for ck in range(_NCH)
    ]
    for cp in copies:
        cp.wait()

    rb = (r // 8) * 8
    cb = (c // 128) * 128

    @pl.when((r >= base) & (r < base + _RPW))
    def _():
        pltpu.sync_copy(onehot_hbm, wbuf)
        pltpu.sync_copy(wbuf, mask_hbm.at[pl.ds(rb, 8), pl.ds(cb, 128)])


_point_mask = functools.partial(
    pl.kernel,
    out_type=(
        jax.ShapeDtypeStruct((_H, _W), jnp.int8),
        jax.ShapeDtypeStruct((16,), jnp.int32),
    ),
    mesh=plsc.VectorSubcoreMesh(core_axis_name="c", subcore_axis_name="s"),
    scratch_types=[
        pltpu.VMEM((_RC, _W), jnp.int8),
        pltpu.VMEM((16,), jnp.int32),
        pltpu.VMEM((16,), jnp.int32),
        pltpu.VMEM((8, 128), jnp.int8),
        pltpu.SemaphoreType.DMA,
    ],
)(_body)


def kernel(operation, action_type, row, col, grid_height, grid_width):
    head = jnp.stack(
        [
            jnp.asarray(operation, jnp.int32),
            jnp.asarray(row, jnp.int32),
            jnp.asarray(col, jnp.int32),
            jnp.asarray(action_type, jnp.int32),
        ]
    )
    params = jnp.concatenate([head, jnp.zeros((12,), jnp.int32)])
    zeros_block = jnp.zeros((_RC, _W), jnp.int8)
    vr = jnp.clip(jnp.asarray(row, jnp.int32), 0, _H - 1)
    vc = jnp.clip(jnp.asarray(col, jnp.int32), 0, _W - 1)
    onehot = (
        (jnp.arange(8, dtype=jnp.int32)[:, None] == (vr % 8))
        & (jnp.arange(128, dtype=jnp.int32)[None, :] == (vc % 128))
    ).astype(jnp.int8)
    mask, scal = _point_mask(params, zeros_block, onehot)
    return (mask, scal[0], scal[3], scal[1], scal[2])
